# Initial kernel scaffold; baseline (speedup 1.0000x reference)
import jax
import jax.numpy as jnp
from jax.experimental import pallas as pl


def kernel(logits):
    raise NotImplementedError("write your pallas kernel here")


# speedup vs baseline: 12.6027x; 12.6027x over previous
import functools

import jax
import jax.numpy as jnp
from jax.experimental import pallas as pl
from jax.experimental.pallas import tpu as pltpu

_MASK_PCT = 1.0
_I32_MIN = -2147483648
_I32_TOP = 0x7FFFFFFF


def _topk_mask_kernel(x_ref, o_ref, key_ref, *, k: int, vocab: int, rows: int):
    x = x_ref[...]
    xb = x + jnp.float32(0.0)
    b = jax.lax.bitcast_convert_type(xb, jnp.int32)
    key = jnp.where(b >= 0, b, b ^ _I32_TOP)
    key_ref[...] = key

    kk = jnp.int32(k)

    def body(i, pfx):
        bbit = 31 - i
        cand_u = pfx | (jnp.int32(1) << bbit)
        cand_s = cand_u ^ _I32_MIN
        cnt = jnp.sum((key_ref[...] >= cand_s).astype(jnp.int32), axis=1,
                      keepdims=True)
        return jnp.where(cnt >= kk, cand_u, pfx)

    pfx = jax.lax.fori_loop(0, 32, body, jnp.zeros((rows, 1), jnp.int32))
    t = pfx ^ _I32_MIN

    key = key_ref[...]
    ge = key >= t
    gt = key > t
    n_ge = jnp.sum(ge.astype(jnp.int32), axis=1, keepdims=True)
    no_boundary_tie = jnp.sum(n_ge) == jnp.int32(rows * k)
    neg_inf = jnp.float32(-jnp.inf)

    @pl.when(no_boundary_tie)
    def _():
        o_ref[...] = jnp.where(ge, neg_inf, x_ref[...])

    @pl.when(jnp.logical_not(no_boundary_tie))
    def _():
        n_gt = jnp.sum(gt.astype(jnp.int32), axis=1, keepdims=True)
        r = kk - n_gt
        eq = ge & jnp.logical_not(gt)
        col = jax.lax.broadcasted_iota(jnp.int32, (rows, vocab), 1)

        def cbody(j, c):
            step = jnp.int32(1) << (17 - j)
            cnt = jnp.sum((eq & (col < (c + step))).astype(jnp.int32),
                          axis=1, keepdims=True)
            return jnp.where(cnt < r, c + step, c)

        c = jax.lax.fori_loop(0, 18, cbody, jnp.zeros((rows, 1), jnp.int32))
        mask = gt | (eq & (col <= c))
        o_ref[...] = jnp.where(mask, neg_inf, x_ref[...])


def kernel(logits):
    rows, vocab = logits.shape
    k = int(vocab * _MASK_PCT / 100)
    block_rows = 8
    grid = rows // block_rows
    f = pl.pallas_call(
        functools.partial(_topk_mask_kernel, k=k, vocab=vocab,
                          rows=block_rows),
        grid=(grid,),
        in_specs=[pl.BlockSpec((block_rows, vocab), lambda i: (i, 0))],
        out_specs=pl.BlockSpec((block_rows, vocab), lambda i: (i, 0)),
        out_shape=jax.ShapeDtypeStruct((rows, vocab), jnp.float32),
        scratch_shapes=[pltpu.VMEM((block_rows, vocab), jnp.int32)],
    )
    return f(logits)


# speedup vs baseline: 14.8476x; 1.1781x over previous
import functools

import jax
import jax.numpy as jnp
from jax.experimental import pallas as pl
from jax.experimental.pallas import tpu as pltpu

_MASK_PCT = 1.0
_I32_MIN = -2147483648
_I32_TOP = 0x7FFFFFFF


def _chunk_bounds(vocab):
    step = 25600
    bounds = list(range(0, vocab, step)) + [vocab]
    return list(zip(bounds[:-1], bounds[1:]))


def _count_ge_i16(ref, cand_i16, chunks):
    cnt = None
    for c0, c1 in chunks:
        m = ref[:, c0:c1] >= cand_i16
        s = jnp.sum(m.astype(jnp.int16), axis=1, keepdims=True).astype(
            jnp.int32)
        cnt = s if cnt is None else cnt + s
    return cnt


def _descent16(ref, kk, rows, chunks):
    def body(i, pfx):
        cand_u = pfx | (jnp.int32(1) << (15 - i))
        cand_i16 = (cand_u ^ 0x8000).astype(jnp.int16)
        cnt = _count_ge_i16(ref, cand_i16, chunks)
        return jnp.where(cnt >= kk, cand_u, pfx)

    return jax.lax.fori_loop(0, 16, body, jnp.zeros((rows, 1), jnp.int32))


def _topk_mask_kernel(x_ref, o_ref, key_ref, hi_ref, lo_ref, *, k: int,
                      vocab: int, rows: int):
    chunks = _chunk_bounds(vocab)
    x = x_ref[...]
    xb = x + jnp.float32(0.0)
    b = jax.lax.bitcast_convert_type(xb, jnp.int32)
    key = jnp.where(b >= 0, b, b ^ _I32_TOP)
    key_ref[...] = key
    key_u = key ^ _I32_MIN
    hi_u = jax.lax.shift_right_logical(key_u, 16)
    hi_ref[...] = (hi_u ^ 0x8000).astype(jnp.int16)

    kk = jnp.int32(k)

    h_u = _descent16(hi_ref, kk, rows, chunks)
    h_i16 = (h_u ^ 0x8000).astype(jnp.int16)

    n_gt_hi = _count_ge_i16(hi_ref, h_i16, chunks) - _count_eq_i16(
        hi_ref, h_i16, chunks)
    k2 = kk - n_gt_hi

    lo_u = key_u & 0xFFFF
    lo_b = (lo_u ^ 0x8000).astype(jnp.int16)
    hi_match = hi_ref[...] == h_i16
    lo_ref[...] = jnp.where(hi_match, lo_b, jnp.int16(-32768))
    l_u = _descent16(lo_ref, k2, rows, chunks)

    t = ((h_u << 16) | l_u) ^ _I32_MIN

    key = key_ref[...]
    ge = key >= t
    gt = key > t
    n_ge = jnp.sum(ge.astype(jnp.int32), axis=1, keepdims=True)
    no_boundary_tie = jnp.sum(n_ge) == jnp.int32(rows * k)
    neg_inf = jnp.float32(-jnp.inf)

    @pl.when(no_boundary_tie)
    def _():
        o_ref[...] = jnp.where(ge, neg_inf, x_ref[...])

    @pl.when(jnp.logical_not(no_boundary_tie))
    def _():
        n_gt = jnp.sum(gt.astype(jnp.int32), axis=1, keepdims=True)
        r = kk - n_gt
        eq = ge & jnp.logical_not(gt)
        col = jax.lax.broadcasted_iota(jnp.int32, (rows, vocab), 1)

        def cbody(j, c):
            step = jnp.int32(1) << (17 - j)
            cnt = jnp.sum((eq & (col < (c + step))).astype(jnp.int32),
                          axis=1, keepdims=True)
            return jnp.where(cnt < r, c + step, c)

        c = jax.lax.fori_loop(0, 18, cbody, jnp.zeros((rows, 1), jnp.int32))
        mask = gt | (eq & (col <= c))
        o_ref[...] = jnp.where(mask, neg_inf, x_ref[...])


def _count_eq_i16(ref, cand_i16, chunks):
    cnt = None
    for c0, c1 in chunks:
        m = ref[:, c0:c1] == cand_i16
        s = jnp.sum(m.astype(jnp.int16), axis=1, keepdims=True).astype(
            jnp.int32)
        cnt = s if cnt is None else cnt + s
    return cnt


def kernel(logits):
    rows, vocab = logits.shape
    k = int(vocab * _MASK_PCT / 100)
    block_rows = 8
    grid = rows // block_rows
    f = pl.pallas_call(
        functools.partial(_topk_mask_kernel, k=k, vocab=vocab,
                          rows=block_rows),
        grid=(grid,),
        in_specs=[pl.BlockSpec((block_rows, vocab), lambda i: (i, 0))],
        out_specs=pl.BlockSpec((block_rows, vocab), lambda i: (i, 0)),
        out_shape=jax.ShapeDtypeStruct((rows, vocab), jnp.float32),
        scratch_shapes=[
            pltpu.VMEM((block_rows, vocab), jnp.int32),
            pltpu.VMEM((block_rows, vocab), jnp.int16),
            pltpu.VMEM((block_rows, vocab), jnp.int16),
        ],
    )
    return f(logits)


# speedup vs baseline: 23.7776x; 1.6014x over previous
import functools

import jax
import jax.numpy as jnp
from jax.experimental import pallas as pl
from jax.experimental.pallas import tpu as pltpu

_MASK_PCT = 1.0
_I32_MIN = -2147483648
_I32_TOP = 0x7FFFFFFF


def _chunk_bounds(vocab):
    step = 24576
    bounds = list(range(0, vocab, step)) + [vocab]
    return list(zip(bounds[:-1], bounds[1:]))


def _fold_sum(m_i16):
    w = m_i16.shape[1]
    while w % 2 == 0 and (w // 2) % 128 == 0:
        h = w // 2
        m_i16 = m_i16[:, :h] + m_i16[:, h:]
        w = h
    return jnp.sum(m_i16.astype(jnp.int32), axis=1, keepdims=True)


def _count_ge_i16(ref, cand_i16, chunks):
    cnt = None
    for c0, c1 in chunks:
        s = _fold_sum((ref[:, c0:c1] >= cand_i16).astype(jnp.int16))
        cnt = s if cnt is None else cnt + s
    return cnt


def _count_gt_i16(ref, cand_i16, chunks):
    cnt = None
    for c0, c1 in chunks:
        s = _fold_sum((ref[:, c0:c1] > cand_i16).astype(jnp.int16))
        cnt = s if cnt is None else cnt + s
    return cnt


def _descent16(ref, kk, rows, chunks):
    def body(i, pfx):
        cand_u = pfx | (jnp.int32(1) << (15 - i))
        cand_i16 = (cand_u ^ 0x8000).astype(jnp.int16)
        cnt = _count_ge_i16(ref, cand_i16, chunks)
        return jnp.where(cnt >= kk, cand_u, pfx)

    return jax.lax.fori_loop(0, 16, body, jnp.zeros((rows, 1), jnp.int32))


def _topk_mask_kernel(x_ref, o_ref, key_ref, hi_ref, lo_ref, *, k: int,
                      vocab: int, rows: int):
    chunks = _chunk_bounds(vocab)
    x = x_ref[...]
    xb = x + jnp.float32(0.0)
    b = jax.lax.bitcast_convert_type(xb, jnp.int32)
    key = jnp.where(b >= 0, b, b ^ _I32_TOP)
    key_ref[...] = key
    key_u = key ^ _I32_MIN
    hi_u = jax.lax.shift_right_logical(key_u, 16)
    hi_ref[...] = (hi_u ^ 0x8000).astype(jnp.int16)

    kk = jnp.int32(k)

    h_u = _descent16(hi_ref, kk, rows, chunks)
    h_i16 = (h_u ^ 0x8000).astype(jnp.int16)

    n_gt_hi = _count_gt_i16(hi_ref, h_i16, chunks)
    k2 = kk - n_gt_hi

    lo_u = key_u & 0xFFFF
    lo_b = (lo_u ^ 0x8000).astype(jnp.int16)
    hi_match = hi_ref[...] == h_i16
    lo_ref[...] = jnp.where(hi_match, lo_b, jnp.int16(-32768))
    l_u = _descent16(lo_ref, k2, rows, chunks)

    t = ((h_u << 16) | l_u) ^ _I32_MIN

    key = key_ref[...]
    ge = key >= t
    gt = key > t
    n_ge = jnp.sum(ge.astype(jnp.int32), axis=1, keepdims=True)
    no_boundary_tie = jnp.sum(n_ge) == jnp.int32(rows * k)
    neg_inf = jnp.float32(-jnp.inf)

    @pl.when(no_boundary_tie)
    def _():
        o_ref[...] = jnp.where(ge, neg_inf, x_ref[...])

    @pl.when(jnp.logical_not(no_boundary_tie))
    def _():
        n_gt = jnp.sum(gt.astype(jnp.int32), axis=1, keepdims=True)
        r = kk - n_gt
        eq = ge & jnp.logical_not(gt)
        col = jax.lax.broadcasted_iota(jnp.int32, (rows, vocab), 1)

        def cbody(j, c):
            step = jnp.int32(1) << (17 - j)
            cnt = jnp.sum((eq & (col < (c + step))).astype(jnp.int32),
                          axis=1, keepdims=True)
            return jnp.where(cnt < r, c + step, c)

        c = jax.lax.fori_loop(0, 18, cbody, jnp.zeros((rows, 1), jnp.int32))
        mask = gt | (eq & (col <= c))
        o_ref[...] = jnp.where(mask, neg_inf, x_ref[...])


def kernel(logits):
    rows, vocab = logits.shape
    k = int(vocab * _MASK_PCT / 100)
    block_rows = 8
    grid = rows // block_rows
    f = pl.pallas_call(
        functools.partial(_topk_mask_kernel, k=k, vocab=vocab,
                          rows=block_rows),
        grid=(grid,),
        in_specs=[pl.BlockSpec((block_rows, vocab), lambda i: (i, 0))],
        out_specs=pl.BlockSpec((block_rows, vocab), lambda i: (i, 0)),
        out_shape=jax.ShapeDtypeStruct((rows, vocab), jnp.float32),
        scratch_shapes=[
            pltpu.VMEM((block_rows, vocab), jnp.int32),
            pltpu.VMEM((block_rows, vocab), jnp.int16),
            pltpu.VMEM((block_rows, vocab), jnp.int16),
        ],
    )
    return f(logits)


# speedup vs baseline: 26.2127x; 1.1024x over previous
import functools

import jax
import jax.numpy as jnp
from jax.experimental import pallas as pl
from jax.experimental.pallas import tpu as pltpu

_MASK_PCT = 1.0
_I32_MIN = -2147483648
_I32_TOP = 0x7FFFFFFF


def _chunk_bounds(vocab):
    step = 24576
    bounds = list(range(0, vocab, step)) + [vocab]
    return list(zip(bounds[:-1], bounds[1:]))


def _fold_sum(m_i16):
    w = m_i16.shape[1]
    while w % 2 == 0 and (w // 2) % 128 == 0:
        h = w // 2
        m_i16 = m_i16[:, :h] + m_i16[:, h:]
        w = h
    return jnp.sum(m_i16.astype(jnp.int32), axis=1, keepdims=True)


def _count_ge_i16(ref, cand_i16, chunks):
    cnt = None
    for c0, c1 in chunks:
        s = _fold_sum((ref[:, c0:c1] >= cand_i16).astype(jnp.int16))
        cnt = s if cnt is None else cnt + s
    return cnt


def _count_gt_i16(ref, cand_i16, chunks):
    cnt = None
    for c0, c1 in chunks:
        s = _fold_sum((ref[:, c0:c1] > cand_i16).astype(jnp.int16))
        cnt = s if cnt is None else cnt + s
    return cnt


def _descent16(ref, kk, rows, chunks):
    def body(i, pfx):
        cand_u = pfx | (jnp.int32(1) << (15 - i))
        cand_i16 = (cand_u ^ 0x8000).astype(jnp.int16)
        cnt = _count_ge_i16(ref, cand_i16, chunks)
        return jnp.where(cnt >= kk, cand_u, pfx)

    return jax.lax.fori_loop(0, 16, body, jnp.zeros((rows, 1), jnp.int32))


def _topk_mask_kernel(x_ref, o_ref, key_ref, hi_ref, lo_ref, *, k: int,
                      vocab: int, rows: int):
    chunks = _chunk_bounds(vocab)
    x = x_ref[...]
    xb = x + jnp.float32(0.0)
    b = jax.lax.bitcast_convert_type(xb, jnp.int32)
    key = jnp.where(b >= 0, b, b ^ _I32_TOP)
    key_ref[...] = key
    key_u = key ^ _I32_MIN
    hi_u = jax.lax.shift_right_logical(key_u, 16)
    hi_ref[...] = (hi_u ^ 0x8000).astype(jnp.int16)

    kk = jnp.int32(k)

    h_u = _descent16(hi_ref, kk, rows, chunks)
    h_i16 = (h_u ^ 0x8000).astype(jnp.int16)

    n_gt_hi = _count_gt_i16(hi_ref, h_i16, chunks)
    k2 = kk - n_gt_hi

    lo_u = key_u & 0xFFFF
    lo_b = (lo_u ^ 0x8000).astype(jnp.int16)
    hi_match = hi_ref[...] == h_i16
    lo_ref[...] = jnp.where(hi_match, lo_b, jnp.int16(-32768))
    l_u = _descent16(lo_ref, k2, rows, chunks)

    t = ((h_u << 16) | l_u) ^ _I32_MIN

    key = key_ref[...]
    ge = key >= t
    gt = key > t
    n_ge = jnp.sum(ge.astype(jnp.int32), axis=1, keepdims=True)
    no_boundary_tie = jnp.sum(n_ge) == jnp.int32(rows * k)
    neg_inf = jnp.float32(-jnp.inf)

    @pl.when(no_boundary_tie)
    def _():
        o_ref[...] = jnp.where(ge, neg_inf, x_ref[...])

    @pl.when(jnp.logical_not(no_boundary_tie))
    def _():
        n_gt = jnp.sum(gt.astype(jnp.int32), axis=1, keepdims=True)
        r = kk - n_gt
        eq = ge & jnp.logical_not(gt)
        col = jax.lax.broadcasted_iota(jnp.int32, (rows, vocab), 1)

        def cbody(j, c):
            step = jnp.int32(1) << (17 - j)
            cnt = jnp.sum((eq & (col < (c + step))).astype(jnp.int32),
                          axis=1, keepdims=True)
            return jnp.where(cnt < r, c + step, c)

        c = jax.lax.fori_loop(0, 18, cbody, jnp.zeros((rows, 1), jnp.int32))
        mask = gt | (eq & (col <= c))
        o_ref[...] = jnp.where(mask, neg_inf, x_ref[...])


def kernel(logits):
    rows, vocab = logits.shape
    k = int(vocab * _MASK_PCT / 100)
    block_rows = 16
    grid = rows // block_rows
    f = pl.pallas_call(
        functools.partial(_topk_mask_kernel, k=k, vocab=vocab,
                          rows=block_rows),
        grid=(grid,),
        in_specs=[pl.BlockSpec((block_rows, vocab), lambda i: (i, 0))],
        out_specs=pl.BlockSpec((block_rows, vocab), lambda i: (i, 0)),
        out_shape=jax.ShapeDtypeStruct((rows, vocab), jnp.float32),
        scratch_shapes=[
            pltpu.VMEM((block_rows, vocab), jnp.int32),
            pltpu.VMEM((block_rows, vocab), jnp.int16),
            pltpu.VMEM((block_rows, vocab), jnp.int16),
        ],
    )
    return f(logits)
